# Initial kernel scaffold; baseline (speedup 1.0000x reference)
import jax
import jax.numpy as jnp
from jax.experimental import pallas as pl


def kernel(batch_mention_emb, mention_type_ids, emb_table, ln_gamma, ln_beta):
    raise NotImplementedError("write your pallas kernel here")


# speedup vs baseline: 3.2244x; 3.2244x over previous
import jax
import jax.numpy as jnp
from jax.experimental import pallas as pl
from jax.experimental.pallas import tpu as pltpu

_EPS = 1e-5


def _fused_body(ids_ref, x_ref, tbl_ref, g_ref, b_ref, o_ref):
    ids = ids_ref[0, 0, :]
    r = ids.shape[0]
    k = tbl_ref.shape[0]
    onehot = (ids[:, None] == jax.lax.broadcasted_iota(jnp.int32, (r, k), 1))
    e = jnp.dot(onehot.astype(jnp.float32), tbl_ref[...],
                preferred_element_type=jnp.float32)
    y = x_ref[...] + e
    mean = jnp.mean(y, axis=1, keepdims=True)
    yc = y - mean
    var = jnp.mean(yc * yc, axis=1, keepdims=True)
    o_ref[...] = yc * jax.lax.rsqrt(var + _EPS) * g_ref[...] + b_ref[...]


def kernel(batch_mention_emb, mention_type_ids, emb_table, ln_gamma, ln_beta):
    b, s, h = batch_mention_emb.shape
    n = b * s
    r = 512
    nblk = n // r
    x = batch_mention_emb.reshape(n, h)
    ids = mention_type_ids.reshape(nblk, 1, r).astype(jnp.int32)
    k = 128
    tbl = jnp.zeros((k, h), emb_table.dtype).at[: emb_table.shape[0]].set(emb_table)
    out = pl.pallas_call(
        _fused_body,
        grid=(nblk,),
        in_specs=[
            pl.BlockSpec((1, 1, r), lambda i: (i, 0, 0)),
            pl.BlockSpec((r, h), lambda i: (i, 0)),
            pl.BlockSpec((k, h), lambda i: (0, 0)),
            pl.BlockSpec((1, h), lambda i: (0, 0)),
            pl.BlockSpec((1, h), lambda i: (0, 0)),
        ],
        out_specs=pl.BlockSpec((r, h), lambda i: (i, 0)),
        out_shape=jax.ShapeDtypeStruct((n, h), jnp.float32),
        compiler_params=pltpu.CompilerParams(dimension_semantics=("arbitrary",)),
    )(ids, x, tbl, ln_gamma.reshape(1, h), ln_beta.reshape(1, h))
    return out.reshape(b, s, h)


# speedup vs baseline: 3.7846x; 1.1737x over previous
import jax
import jax.numpy as jnp
from jax.experimental import pallas as pl
from jax.experimental.pallas import tpu as pltpu

_EPS = 1e-5


def _fused_body(ids_ref, x_ref, tbl_ref, g_ref, b_ref, o_ref):
    ids = ids_ref[0, 0, :]
    r = ids.shape[0]
    k = tbl_ref.shape[0]
    onehot = (ids[:, None] == jax.lax.broadcasted_iota(jnp.int32, (r, k), 1))
    e = jnp.dot(onehot.astype(jnp.float32), tbl_ref[...],
                preferred_element_type=jnp.float32)
    y = x_ref[...] + e
    mean = jnp.mean(y, axis=1, keepdims=True)
    yc = y - mean
    var = jnp.mean(yc * yc, axis=1, keepdims=True)
    o_ref[...] = yc * jax.lax.rsqrt(var + _EPS) * g_ref[...] + b_ref[...]


def kernel(batch_mention_emb, mention_type_ids, emb_table, ln_gamma, ln_beta):
    b, s, h = batch_mention_emb.shape
    n = b * s
    r = 1024
    nblk = n // r
    x = batch_mention_emb.reshape(n, h)
    ids = mention_type_ids.reshape(nblk, 1, r).astype(jnp.int32)
    k = 128
    tbl = jnp.zeros((k, h), emb_table.dtype).at[: emb_table.shape[0]].set(emb_table)
    out = pl.pallas_call(
        _fused_body,
        grid=(nblk,),
        in_specs=[
            pl.BlockSpec((1, 1, r), lambda i: (i, 0, 0)),
            pl.BlockSpec((r, h), lambda i: (i, 0)),
            pl.BlockSpec((k, h), lambda i: (0, 0)),
            pl.BlockSpec((1, h), lambda i: (0, 0)),
            pl.BlockSpec((1, h), lambda i: (0, 0)),
        ],
        out_specs=pl.BlockSpec((r, h), lambda i: (i, 0)),
        out_shape=jax.ShapeDtypeStruct((n, h), jnp.float32),
        compiler_params=pltpu.CompilerParams(dimension_semantics=("arbitrary",)),
    )(ids, x, tbl, ln_gamma.reshape(1, h), ln_beta.reshape(1, h))
    return out.reshape(b, s, h)


# speedup vs baseline: 3.9666x; 1.0481x over previous
import jax
import jax.numpy as jnp
from jax.experimental import pallas as pl
from jax.experimental.pallas import tpu as pltpu

_EPS = 1e-5


def _fused_body(ids_ref, x_ref, tbl_ref, g_ref, b_ref, o_ref):
    ids = ids_ref[0, 0, :]
    r = ids.shape[0]
    k = tbl_ref.shape[0]
    onehot = (ids[:, None] == jax.lax.broadcasted_iota(jnp.int32, (r, k), 1))
    e = jnp.dot(onehot.astype(jnp.float32), tbl_ref[...],
                preferred_element_type=jnp.float32)
    y = x_ref[...] + e
    mean = jnp.mean(y, axis=1, keepdims=True)
    yc = y - mean
    var = jnp.mean(yc * yc, axis=1, keepdims=True)
    o_ref[...] = yc * jax.lax.rsqrt(var + _EPS) * g_ref[...] + b_ref[...]


def kernel(batch_mention_emb, mention_type_ids, emb_table, ln_gamma, ln_beta):
    b, s, h = batch_mention_emb.shape
    n = b * s
    r = 2048
    nblk = n // r
    x = batch_mention_emb.reshape(n, h)
    ids = mention_type_ids.reshape(nblk, 1, r).astype(jnp.int32)
    k = 128
    tbl = jnp.zeros((k, h), emb_table.dtype).at[: emb_table.shape[0]].set(emb_table)
    out = pl.pallas_call(
        _fused_body,
        grid=(nblk,),
        in_specs=[
            pl.BlockSpec((1, 1, r), lambda i: (i, 0, 0)),
            pl.BlockSpec((r, h), lambda i: (i, 0)),
            pl.BlockSpec((k, h), lambda i: (0, 0)),
            pl.BlockSpec((1, h), lambda i: (0, 0)),
            pl.BlockSpec((1, h), lambda i: (0, 0)),
        ],
        out_specs=pl.BlockSpec((r, h), lambda i: (i, 0)),
        out_shape=jax.ShapeDtypeStruct((n, h), jnp.float32),
        compiler_params=pltpu.CompilerParams(dimension_semantics=("arbitrary",)),
    )(ids, x, tbl, ln_gamma.reshape(1, h), ln_beta.reshape(1, h))
    return out.reshape(b, s, h)
